# baseline (speedup 1.0000x reference)
import functools

import jax
import jax.numpy as jnp
from jax import lax
from jax.experimental import pallas as pl
from jax.experimental.pallas import tpu as pltpu
from jax.experimental.pallas import tpu_sc as plsc

_CHANNEL_INDICES = tuple(range(64))

_B, _C, _D = 1024, 64, 768


def _coalesce_runs(indices):
    runs = []
    out_start, src_start, length = 0, indices[0], 1
    for pos in range(1, len(indices)):
        if indices[pos] == indices[pos - 1] + 1:
            length += 1
        else:
            runs.append((out_start, src_start, length))
            out_start, src_start, length = pos, indices[pos], 1
    runs.append((out_start, src_start, length))
    return runs


_RUNS = _coalesce_runs(_CHANNEL_INDICES)

_NUM_WORKERS = 32
_ROWS_PER_WORKER = _B // _NUM_WORKERS


@functools.partial(
    pl.kernel,
    out_type=jax.ShapeDtypeStruct((_B, _C * _D), jnp.float32),
    mesh=plsc.VectorSubcoreMesh(core_axis_name="c", subcore_axis_name="s"),
)
def _sc_gather_copy(x_hbm, out_hbm):
    wid = lax.axis_index("s") * 2 + lax.axis_index("c")
    base = wid * _ROWS_PER_WORKER
    row_slc = pl.ds(base, _ROWS_PER_WORKER)
    for out_start, src_start, length in _RUNS:
        pltpu.sync_copy(
            x_hbm.at[row_slc, pl.ds(src_start * _D, length * _D)],
            out_hbm.at[row_slc, pl.ds(out_start * _D, length * _D)],
        )


def kernel(x):
    out = _sc_gather_copy(x.reshape(_B, _C * _D))
    return out.reshape(_B, _C, _D)


# speedup vs baseline: 14.4200x; 14.4200x over previous
import functools

import jax
import jax.numpy as jnp
from jax import lax
from jax.experimental import pallas as pl
from jax.experimental.pallas import tpu as pltpu
from jax.experimental.pallas import tpu_sc as plsc

_CHANNEL_INDICES = tuple(range(64))

_B, _C, _D = 1024, 64, 768


def _coalesce_runs(indices):
    runs = []
    out_start, src_start, length = 0, indices[0], 1
    for pos in range(1, len(indices)):
        if indices[pos] == indices[pos - 1] + 1:
            length += 1
        else:
            runs.append((out_start, src_start, length))
            out_start, src_start, length = pos, indices[pos], 1
    runs.append((out_start, src_start, length))
    return runs


_RUNS = _coalesce_runs(_CHANNEL_INDICES)

_NUM_WORKERS = 32
_ROWS_PER_WORKER = _B // _NUM_WORKERS
_ROW = _C * _D


@functools.partial(
    pl.kernel,
    out_type=jax.ShapeDtypeStruct((_B, _ROW), jnp.float32),
    mesh=plsc.VectorSubcoreMesh(core_axis_name="c", subcore_axis_name="s"),
    scratch_types=[
        pltpu.VMEM((2, _ROW), jnp.float32),
        pltpu.SemaphoreType.DMA,
        pltpu.SemaphoreType.DMA,
    ],
)
def _sc_gather_copy(x_hbm, out_hbm, buf, in_sem, out_sem):
    wid = lax.axis_index("s") * 2 + lax.axis_index("c")
    base = wid * _ROWS_PER_WORKER

    def load(i):
        slot = buf.at[i % 2]
        return [
            pltpu.make_async_copy(
                x_hbm.at[base + i, pl.ds(src * _D, ln * _D)],
                slot.at[pl.ds(dst * _D, ln * _D)],
                in_sem,
            )
            for dst, src, ln in _RUNS
        ]

    def store(i):
        return pltpu.make_async_copy(buf.at[i % 2], out_hbm.at[base + i], out_sem)

    for c in load(0):
        c.start()
    for i in range(_ROWS_PER_WORKER):
        for c in load(i):
            c.wait()
        if i + 1 < _ROWS_PER_WORKER:
            if i >= 1:
                store(i - 1).wait()
            for c in load(i + 1):
                c.start()
        store(i).start()
    store(_ROWS_PER_WORKER - 2).wait()
    store(_ROWS_PER_WORKER - 1).wait()


def kernel(x):
    out = _sc_gather_copy(x.reshape(_B, _ROW))
    return out.reshape(_B, _C, _D)


# speedup vs baseline: 40.1740x; 2.7860x over previous
import functools

import jax
import jax.numpy as jnp
from jax import lax
from jax.experimental import pallas as pl
from jax.experimental.pallas import tpu as pltpu
from jax.experimental.pallas import tpu_sc as plsc

_CHANNEL_INDICES = tuple(range(64))

_B, _C, _D = 1024, 64, 768


def _coalesce_runs(indices):
    runs = []
    out_start, src_start, length = 0, indices[0], 1
    for pos in range(1, len(indices)):
        if indices[pos] == indices[pos - 1] + 1:
            length += 1
        else:
            runs.append((out_start, src_start, length))
            out_start, src_start, length = pos, indices[pos], 1
    runs.append((out_start, src_start, length))
    return runs


_RUNS = _coalesce_runs(_CHANNEL_INDICES)

_NUM_WORKERS = 32
_ROWS_PER_WORKER = _B // _NUM_WORKERS

_NBUF = 2
_LOOK = 1
_CB = _C
_NCHUNK = _ROWS_PER_WORKER * (_C // _CB)


def _chunk_coords(i):
    per = _C // _CB
    return i // per, (i % per) * _CB


@functools.partial(
    pl.kernel,
    out_type=jax.ShapeDtypeStruct((_B, _C, _D), jnp.float32),
    mesh=plsc.VectorSubcoreMesh(core_axis_name="c", subcore_axis_name="s"),
    scratch_types=[
        pltpu.VMEM((_NBUF, _CB, _D), jnp.float32),
        [pltpu.SemaphoreType.DMA] * _NBUF,
        [pltpu.SemaphoreType.DMA] * _NBUF,
    ],
)
def _sc_gather_copy(x_hbm, out_hbm, buf, in_sems, out_sems):
    wid = lax.axis_index("s") * 2 + lax.axis_index("c")
    base = wid * _ROWS_PER_WORKER

    def load(i):
        row, c0 = _chunk_coords(i)
        s = i % _NBUF
        copies = []
        for out_start, src_start, length in _RUNS:
            lo = max(out_start, c0)
            hi = min(out_start + length, c0 + _CB)
            if lo >= hi:
                continue
            src_lo = src_start + (lo - out_start)
            copies.append(
                pltpu.make_async_copy(
                    x_hbm.at[base + row, pl.ds(src_lo, hi - lo)],
                    buf.at[s, pl.ds(lo - c0, hi - lo)],
                    in_sems[s],
                )
            )
        return copies

    def store(i):
        row, c0 = _chunk_coords(i)
        s = i % _NBUF
        return pltpu.make_async_copy(
            buf.at[s], out_hbm.at[base + row, pl.ds(c0, _CB)], out_sems[s]
        )

    store_waited = set()

    def wait_store(k):
        if 0 <= k < _NCHUNK and k not in store_waited:
            store(k).wait()
            store_waited.add(k)

    for j in range(_LOOK):
        for cpy in load(j):
            cpy.start()
    for i in range(_NCHUNK):
        for cpy in load(i):
            cpy.wait()
        store(i).start()
        nxt = i + _LOOK
        if nxt < _NCHUNK:
            wait_store(nxt - _NBUF)
            for cpy in load(nxt):
                cpy.start()
    for k in range(_NCHUNK):
        wait_store(k)


def kernel(x):
    return _sc_gather_copy(x)
